# baseline (speedup 1.0000x reference)
import functools

import jax
import jax.numpy as jnp
from jax import lax
from jax.experimental import pallas as pl
from jax.experimental.pallas import tpu as pltpu, tpu_sc as plsc

N = 10000
E = 160000
D = 256
DH = 128
DE = 16

N_PAD = 10240
E_PAD = 160000
CHUNK = 40
NSLOT = 3
N_TILES = 16
EDGES_PER_TILE = E_PAD // N_TILES
CHUNKS_PER_TILE = EDGES_PER_TILE // CHUNK
AGG_ROWS = 10112
ROWS_PER_TILE = AGG_ROWS // N_TILES

_NB = N_PAD // 512
_EROWS = 8000
_EB = E_PAD // _EROWS


def _h_body(x_ref, w_ref, h2_ref):
    h2_ref[...] = jnp.dot(x_ref[...], w_ref[...],
                          preferred_element_type=jnp.float32)


def _ep_body(ea_ref, we_ref, b_ref, ep_ref, pol_ref):
    c = pl.program_id(1)
    ep_ref[...] = jnp.dot(ea_ref[...], we_ref[...],
                          preferred_element_type=jnp.float32) + b_ref[pl.ds(c, 1), :]
    pol_ref[...] = jnp.broadcast_to(jnp.tanh(ea_ref[:, :1]), (_EROWS, 16))


def _out_body(x_ref, w_ref, alo_ref, ahi_ref, o_ref):
    acc = jnp.dot(x_ref[...], w_ref[...], preferred_element_type=jnp.float32)
    agg = jnp.concatenate([alo_ref[...], ahi_ref[...]], axis=1)
    o_ref[...] = jnp.maximum(acc + agg, 0.0)


def _sc_scatter_body(h2, sdp, dst, polh, ep2, zeros, agg_out,
                     sdp_v0, sdp_v1, sdp_v2, dst_v0, dst_v1, dst_v2,
                     pol_v0, pol_v1, pol_v2, ep_buf0, ep_buf1, ep_buf2,
                     g_buf0, g_buf1, g_buf2, agg_sh,
                     sem_s0, sem_s1, sem_s2, sem_b0, sem_b1, sem_b2):
    c = lax.axis_index("c")
    s = lax.axis_index("s")
    row0 = s * ROWS_PER_TILE

    pltpu.sync_copy(zeros, agg_sh.at[pl.ds(row0, ROWS_PER_TILE)])
    plsc.subcore_barrier()

    ep_base0 = c * E_PAD
    edge0 = s * EDGES_PER_TILE
    LAST = CHUNKS_PER_TILE - 1
    sdp_v = (sdp_v0, sdp_v1, sdp_v2)
    dst_v = (dst_v0, dst_v1, dst_v2)
    pol_v = (pol_v0, pol_v1, pol_v2)
    ep_buf = (ep_buf0, ep_buf1, ep_buf2)
    g_buf = (g_buf0, g_buf1, g_buf2)
    sem_s = (sem_s0, sem_s1, sem_s2)
    sem_b = (sem_b0, sem_b1, sem_b2)
    rec0 = (c * N_TILES + s) * CHUNKS_PER_TILE * 128

    def idx_copies(k, slot):
        base = edge0 + k * CHUNK
        return (
            (sdp.at[pl.ds(rec0 + k * 128, 128)], sdp_v[slot]),
            (dst.at[pl.ds(base, CHUNK)], dst_v[slot]),
            (polh.at[pl.ds(base, CHUNK)], pol_v[slot]),
        )

    def issue_idx(k, slot):
        for a, b in idx_copies(k, slot):
            pltpu.async_copy(a, b, sem_s[slot])

    def wait_idx(k, slot):
        for a, b in idx_copies(k, slot):
            pltpu.make_async_copy(a, b, sem_s[slot]).wait()

    def big_copies(k, slot):
        base = edge0 + k * CHUNK
        half = CHUNK // 2
        return (
            (ep2.at[pl.ds(ep_base0 + base, CHUNK)], ep_buf[slot]),
            (h2.at[sdp_v[slot].at[pl.ds(0, half)]],
             g_buf[slot].at[pl.ds(0, half)]),
            (h2.at[sdp_v[slot].at[pl.ds(24, half)]],
             g_buf[slot].at[pl.ds(half, half)]),
        )

    def issue_big(k, slot):
        for a, b in big_copies(k, slot):
            pltpu.async_copy(a, b, sem_b[slot])

    def wait_big(k, slot):
        for a, b in big_copies(k, slot):
            pltpu.make_async_copy(a, b, sem_b[slot]).wait()

    def compute(slot):
        gb = g_buf[slot]
        eb = ep_buf[slot]
        pb = pol_v[slot]

        def edge_body(j2, _):
            for t in range(2):
                j = j2 * 2 + t
                pv = pb[j, :]
                for r in range(DH // 16):
                    sl = pl.ds(r * 16, 16)
                    gb[j, sl] = jnp.maximum(gb[j, sl] + eb[j, sl], 0.0) * pv
            return 0

        lax.fori_loop(0, CHUNK // 2, edge_body, 0)

    def chunk_step(k, s0, s1, s2):
        k2 = jnp.minimum(k + 2, LAST)
        wait_idx(k2, s2)
        issue_big(k2, s2)
        wait_big(k, s0)
        compute(s0)
        pltpu.sync_copy(g_buf[s0], agg_sh.at[dst_v[s0]], add=True)
        issue_idx(jnp.minimum(k + 3, LAST), s0)

    issue_idx(0, 0)
    issue_idx(1, 1)
    wait_idx(0, 0)
    issue_big(0, 0)
    wait_idx(1, 1)
    issue_big(1, 1)
    issue_idx(2, 2)

    def tri_body(j, _):
        chunk_step(3 * j, 0, 1, 2)
        chunk_step(3 * j + 1, 1, 2, 0)
        chunk_step(3 * j + 2, 2, 0, 1)
        return 0

    lax.fori_loop(0, CHUNKS_PER_TILE // 3, tri_body, 0)
    chunk_step(CHUNKS_PER_TILE - 1, 0, 1, 2)

    wait_big(LAST, 1)
    wait_big(LAST, 2)
    wait_idx(LAST, 0)

    plsc.subcore_barrier()
    pltpu.sync_copy(agg_sh.at[pl.ds(row0, ROWS_PER_TILE)],
                    agg_out.at[pl.ds(c * N_PAD + row0, ROWS_PER_TILE)])


@functools.cache
def _make_sc_scatter():
    mesh = plsc.VectorSubcoreMesh(core_axis_name="c", subcore_axis_name="s",
                                  num_cores=2, num_subcores=N_TILES)
    return pl.kernel(
        _sc_scatter_body,
        out_type=jax.ShapeDtypeStruct((2 * N_PAD, DH), jnp.float32),
        mesh=mesh,
        scratch_types=(
            [pltpu.VMEM((128,), jnp.int32) for _ in range(NSLOT)]
            + [pltpu.VMEM((CHUNK,), jnp.int32) for _ in range(NSLOT)]
            + [pltpu.VMEM((CHUNK, 16), jnp.float32) for _ in range(NSLOT)]
            + [pltpu.VMEM((CHUNK, DH), jnp.float32) for _ in range(NSLOT)]
            + [pltpu.VMEM((CHUNK, DH), jnp.float32) for _ in range(NSLOT)]
            + [pltpu.VMEM_SHARED((AGG_ROWS, DH), jnp.float32)]
            + [pltpu.SemaphoreType.DMA for _ in range(2 * NSLOT)]
        ),
    )


def kernel(x, edge_index, edge_attr, W_msg, W_edge, W_self, b):
    src = edge_index[0]
    dst = edge_index[1]

    x_pad = jnp.concatenate(
        [x, jnp.zeros((N_PAD - N, D), jnp.float32)], axis=0)
    src2 = jnp.concatenate([src, src + N_PAD])
    nrec = 2 * N_TILES * CHUNKS_PER_TILE
    srcr = src2.reshape(2, N_TILES, CHUNKS_PER_TILE, CHUNK)
    dstr = jnp.broadcast_to(
        dst.reshape(1, N_TILES, CHUNKS_PER_TILE, CHUNK),
        (2, N_TILES, CHUNKS_PER_TILE, CHUNK))
    z4 = jnp.zeros((2, N_TILES, CHUNKS_PER_TILE, 4), jnp.int32)
    z40 = jnp.zeros((2, N_TILES, CHUNKS_PER_TILE, 128 - 2 * CHUNK - 8),
                    jnp.int32)
    half = CHUNK // 2
    sdp = jnp.concatenate(
        [srcr[..., :half], z4, srcr[..., half:], z4, dstr, z40],
        axis=-1).reshape(nrec * 128)
    b2 = b.reshape(2, DH)
    zeros = jnp.zeros((ROWS_PER_TILE, DH), jnp.float32)

    h2 = pl.pallas_call(
        _h_body,
        grid=(_NB, 2),
        in_specs=[
            pl.BlockSpec((512, D), lambda i, c: (i, 0)),
            pl.BlockSpec((D, DH), lambda i, c: (0, c)),
        ],
        out_specs=pl.BlockSpec((512, DH), lambda i, c: (c * _NB + i, 0)),
        out_shape=jax.ShapeDtypeStruct((2 * N_PAD, DH), jnp.float32),
    )(x_pad, W_msg)

    ep2, pol2d = pl.pallas_call(
        _ep_body,
        grid=(_EB, 2),
        in_specs=[
            pl.BlockSpec((_EROWS, DE), lambda i, c: (i, 0)),
            pl.BlockSpec((DE, DH), lambda i, c: (0, c)),
            pl.BlockSpec((2, DH), lambda i, c: (0, 0)),
        ],
        out_specs=[
            pl.BlockSpec((_EROWS, DH), lambda i, c: (c * _EB + i, 0)),
            pl.BlockSpec((_EROWS, 16), lambda i, c: (i, 0)),
        ],
        out_shape=[
            jax.ShapeDtypeStruct((2 * E_PAD, DH), jnp.float32),
            jax.ShapeDtypeStruct((E_PAD, 16), jnp.float32),
        ],
    )(edge_attr, W_edge, b2)

    agg2 = _make_sc_scatter()(h2, sdp, dst, pol2d, ep2, zeros)

    out = pl.pallas_call(
        _out_body,
        grid=(_NB,),
        in_specs=[
            pl.BlockSpec((512, D), lambda i: (i, 0)),
            pl.BlockSpec((D, D), lambda i: (0, 0)),
            pl.BlockSpec((512, DH), lambda i: (i, 0)),
            pl.BlockSpec((512, DH), lambda i: (_NB + i, 0)),
        ],
        out_specs=pl.BlockSpec((512, D), lambda i: (i, 0)),
        out_shape=jax.ShapeDtypeStruct((N_PAD, D), jnp.float32),
    )(x_pad, W_self, agg2, agg2)

    return out[:N]


# speedup vs baseline: 1.0883x; 1.0883x over previous
import functools

import jax
import jax.numpy as jnp
from jax import lax
from jax.experimental import pallas as pl
from jax.experimental.pallas import tpu as pltpu, tpu_sc as plsc

N = 10000
E = 160000
D = 256
DH = 128
DE = 16

N_PAD = 10240
E_PAD = 160000
CHUNK = 40
NSLOT = 3
N_TILES = 16
EDGES_PER_TILE = E_PAD // N_TILES
CHUNKS_PER_TILE = EDGES_PER_TILE // CHUNK
AGG_ROWS = 10112
ROWS_PER_TILE = AGG_ROWS // N_TILES

_NB = N_PAD // 512
_EROWS = 16000
_EB = E_PAD // _EROWS


def _h_body(x_ref, w_ref, h2_ref):
    h2_ref[...] = jnp.dot(x_ref[...], w_ref[...],
                          preferred_element_type=jnp.float32)


def _ep_body(ea_ref, we_ref, b_ref, ep_ref, pol_ref):
    c = pl.program_id(1)
    ep_ref[...] = jnp.dot(ea_ref[...], we_ref[...],
                          preferred_element_type=jnp.float32) + b_ref[pl.ds(c, 1), :]
    pol_ref[...] = jnp.broadcast_to(jnp.tanh(ea_ref[:, :1]), (_EROWS, 16))


def _out_body(x_ref, w_ref, alo_ref, ahi_ref, o_ref):
    acc = jnp.dot(x_ref[...], w_ref[...], preferred_element_type=jnp.float32)
    agg = jnp.concatenate([alo_ref[...], ahi_ref[...]], axis=1)
    o_ref[...] = jnp.maximum(acc + agg, 0.0)


def _sc_scatter_body(h2, src2, dst, polh, ep2, zeros, agg_out,
                     src_v0, src_v1, src_v2, dst_v0, dst_v1, dst_v2,
                     pol_v0, pol_v1, pol_v2, ep_buf0, ep_buf1, ep_buf2,
                     g_buf0, g_buf1, g_buf2, agg_sh,
                     sem_s0, sem_s1, sem_s2, sem_b0, sem_b1, sem_b2):
    c = lax.axis_index("c")
    s = lax.axis_index("s")
    row0 = s * ROWS_PER_TILE

    pltpu.sync_copy(zeros, agg_sh.at[pl.ds(row0, ROWS_PER_TILE)])
    plsc.subcore_barrier()

    ep_base0 = c * E_PAD
    edge0 = s * EDGES_PER_TILE
    LAST = CHUNKS_PER_TILE - 1
    src_v = (src_v0, src_v1, src_v2)
    dst_v = (dst_v0, dst_v1, dst_v2)
    pol_v = (pol_v0, pol_v1, pol_v2)
    ep_buf = (ep_buf0, ep_buf1, ep_buf2)
    g_buf = (g_buf0, g_buf1, g_buf2)
    sem_s = (sem_s0, sem_s1, sem_s2)
    sem_b = (sem_b0, sem_b1, sem_b2)

    def idx_copies(k, slot):
        base = edge0 + k * CHUNK
        return (
            (src2.at[pl.ds(c * E_PAD + base, CHUNK)], src_v[slot]),
            (dst.at[pl.ds(base, CHUNK)], dst_v[slot]),
            (polh.at[pl.ds(base, CHUNK)], pol_v[slot]),
        )

    def issue_idx(k, slot):
        for a, b in idx_copies(k, slot):
            pltpu.async_copy(a, b, sem_s[slot])

    def wait_idx(k, slot):
        for a, b in idx_copies(k, slot):
            pltpu.make_async_copy(a, b, sem_s[slot]).wait()

    def big_copies(k, slot):
        base = edge0 + k * CHUNK
        return (
            (ep2.at[pl.ds(ep_base0 + base, CHUNK)], ep_buf[slot]),
            (h2.at[src_v[slot]], g_buf[slot]),
        )

    def issue_big(k, slot):
        for a, b in big_copies(k, slot):
            pltpu.async_copy(a, b, sem_b[slot])

    def wait_big(k, slot):
        for a, b in big_copies(k, slot):
            pltpu.make_async_copy(a, b, sem_b[slot]).wait()

    def compute(slot):
        gb = g_buf[slot]
        eb = ep_buf[slot]
        pb = pol_v[slot]

        def edge_body(j2, _):
            for t in range(2):
                j = j2 * 2 + t
                pv = pb[j, :]
                for r in range(DH // 16):
                    sl = pl.ds(r * 16, 16)
                    gb[j, sl] = jnp.maximum(gb[j, sl] + eb[j, sl], 0.0) * pv
            return 0

        lax.fori_loop(0, CHUNK // 2, edge_body, 0)

    def chunk_step(k, s0, s1, s2):
        k2 = jnp.minimum(k + 2, LAST)
        wait_idx(k2, s2)
        issue_big(k2, s2)
        wait_big(k, s0)
        compute(s0)
        pltpu.sync_copy(g_buf[s0], agg_sh.at[dst_v[s0]], add=True)
        issue_idx(jnp.minimum(k + 3, LAST), s0)

    issue_idx(0, 0)
    issue_idx(1, 1)
    wait_idx(0, 0)
    issue_big(0, 0)
    wait_idx(1, 1)
    issue_big(1, 1)
    issue_idx(2, 2)

    def tri_body(j, _):
        chunk_step(3 * j, 0, 1, 2)
        chunk_step(3 * j + 1, 1, 2, 0)
        chunk_step(3 * j + 2, 2, 0, 1)
        return 0

    lax.fori_loop(0, CHUNKS_PER_TILE // 3, tri_body, 0)
    chunk_step(CHUNKS_PER_TILE - 1, 0, 1, 2)

    wait_big(LAST, 1)
    wait_big(LAST, 2)
    wait_idx(LAST, 0)

    plsc.subcore_barrier()
    pltpu.sync_copy(agg_sh.at[pl.ds(row0, ROWS_PER_TILE)],
                    agg_out.at[pl.ds(c * N_PAD + row0, ROWS_PER_TILE)])


@functools.cache
def _make_sc_scatter():
    mesh = plsc.VectorSubcoreMesh(core_axis_name="c", subcore_axis_name="s",
                                  num_cores=2, num_subcores=N_TILES)
    return pl.kernel(
        _sc_scatter_body,
        out_type=jax.ShapeDtypeStruct((2 * N_PAD, DH), jnp.float32),
        mesh=mesh,
        scratch_types=(
            [pltpu.VMEM((CHUNK,), jnp.int32) for _ in range(NSLOT)]
            + [pltpu.VMEM((CHUNK,), jnp.int32) for _ in range(NSLOT)]
            + [pltpu.VMEM((CHUNK, 16), jnp.float32) for _ in range(NSLOT)]
            + [pltpu.VMEM((CHUNK, DH), jnp.float32) for _ in range(NSLOT)]
            + [pltpu.VMEM((CHUNK, DH), jnp.float32) for _ in range(NSLOT)]
            + [pltpu.VMEM_SHARED((AGG_ROWS, DH), jnp.float32)]
            + [pltpu.SemaphoreType.DMA for _ in range(2 * NSLOT)]
        ),
    )


def kernel(x, edge_index, edge_attr, W_msg, W_edge, W_self, b):
    src = edge_index[0]
    dst = edge_index[1]

    x_pad = jnp.concatenate(
        [x, jnp.zeros((N_PAD - N, D), jnp.float32)], axis=0)
    src2 = jnp.concatenate([src, src + N_PAD])
    dst_pad = dst
    b2 = b.reshape(2, DH)
    zeros = jnp.zeros((ROWS_PER_TILE, DH), jnp.float32)

    h2 = pl.pallas_call(
        _h_body,
        grid=(_NB, 2),
        in_specs=[
            pl.BlockSpec((512, D), lambda i, c: (i, 0)),
            pl.BlockSpec((D, DH), lambda i, c: (0, c)),
        ],
        out_specs=pl.BlockSpec((512, DH), lambda i, c: (c * _NB + i, 0)),
        out_shape=jax.ShapeDtypeStruct((2 * N_PAD, DH), jnp.float32),
    )(x_pad, W_msg)

    ep2, pol2d = pl.pallas_call(
        _ep_body,
        grid=(_EB, 2),
        in_specs=[
            pl.BlockSpec((_EROWS, DE), lambda i, c: (i, 0)),
            pl.BlockSpec((DE, DH), lambda i, c: (0, c)),
            pl.BlockSpec((2, DH), lambda i, c: (0, 0)),
        ],
        out_specs=[
            pl.BlockSpec((_EROWS, DH), lambda i, c: (c * _EB + i, 0)),
            pl.BlockSpec((_EROWS, 16), lambda i, c: (i, 0)),
        ],
        out_shape=[
            jax.ShapeDtypeStruct((2 * E_PAD, DH), jnp.float32),
            jax.ShapeDtypeStruct((E_PAD, 16), jnp.float32),
        ],
    )(edge_attr, W_edge, b2)

    agg2 = _make_sc_scatter()(h2, src2, dst_pad, pol2d, ep2, zeros)

    out = pl.pallas_call(
        _out_body,
        grid=(_NB,),
        in_specs=[
            pl.BlockSpec((512, D), lambda i: (i, 0)),
            pl.BlockSpec((D, D), lambda i: (0, 0)),
            pl.BlockSpec((512, DH), lambda i: (i, 0)),
            pl.BlockSpec((512, DH), lambda i: (_NB + i, 0)),
        ],
        out_specs=pl.BlockSpec((512, D), lambda i: (i, 0)),
        out_shape=jax.ShapeDtypeStruct((N_PAD, D), jnp.float32),
    )(x_pad, W_self, agg2, agg2)

    return out[:N]


# speedup vs baseline: 1.1302x; 1.0385x over previous
import functools

import jax
import jax.numpy as jnp
from jax import lax
from jax.experimental import pallas as pl
from jax.experimental.pallas import tpu as pltpu, tpu_sc as plsc

N = 10000
E = 160000
D = 256
DH = 128
DE = 16

N_PAD = 10240
E_PAD = 160000
CHUNK = 40
NSLOT = 3
N_TILES = 16
EDGES_PER_TILE = E_PAD // N_TILES
CHUNKS_PER_TILE = EDGES_PER_TILE // CHUNK
AGG_ROWS = 10112
ROWS_PER_TILE = AGG_ROWS // N_TILES

_NROWS = 2048
_NB = N_PAD // _NROWS
_EROWS = 16000
_EB = E_PAD // _EROWS


def _h_body(x_ref, w_ref, h2_ref):
    h2_ref[...] = jnp.dot(x_ref[...], w_ref[...],
                          preferred_element_type=jnp.float32)


def _ep_body(ea_ref, we_ref, b_ref, ep_ref, pol_ref):
    c = pl.program_id(1)
    ep_ref[...] = jnp.dot(ea_ref[...], we_ref[...],
                          preferred_element_type=jnp.float32) + b_ref[pl.ds(c, 1), :]
    pol_ref[...] = jnp.broadcast_to(jnp.tanh(ea_ref[:, :1]), (_EROWS, 16))


def _out_body(x_ref, w_ref, alo_ref, ahi_ref, o_ref):
    acc = jnp.dot(x_ref[...], w_ref[...], preferred_element_type=jnp.float32)
    agg = jnp.concatenate([alo_ref[...], ahi_ref[...]], axis=1)
    o_ref[...] = jnp.maximum(acc + agg, 0.0)


def _sc_scatter_body(h2, src2, dst, polh, ep2, zeros, agg_out,
                     src_v0, src_v1, src_v2, dst_v0, dst_v1, dst_v2,
                     pol_v0, pol_v1, pol_v2, ep_buf0, ep_buf1, ep_buf2,
                     g_buf0, g_buf1, g_buf2, agg_sh,
                     sem_s0, sem_s1, sem_s2, sem_b0, sem_b1, sem_b2):
    c = lax.axis_index("c")
    s = lax.axis_index("s")
    row0 = s * ROWS_PER_TILE

    pltpu.sync_copy(zeros, agg_sh.at[pl.ds(row0, ROWS_PER_TILE)])
    plsc.subcore_barrier()

    ep_base0 = c * E_PAD
    edge0 = s * EDGES_PER_TILE
    LAST = CHUNKS_PER_TILE - 1
    src_v = (src_v0, src_v1, src_v2)
    dst_v = (dst_v0, dst_v1, dst_v2)
    pol_v = (pol_v0, pol_v1, pol_v2)
    ep_buf = (ep_buf0, ep_buf1, ep_buf2)
    g_buf = (g_buf0, g_buf1, g_buf2)
    sem_s = (sem_s0, sem_s1, sem_s2)
    sem_b = (sem_b0, sem_b1, sem_b2)

    def idx_copies(k, slot):
        base = edge0 + k * CHUNK
        return (
            (src2.at[pl.ds(c * E_PAD + base, CHUNK)], src_v[slot]),
            (dst.at[pl.ds(base, CHUNK)], dst_v[slot]),
            (polh.at[pl.ds(base, CHUNK)], pol_v[slot]),
        )

    def issue_idx(k, slot):
        for a, b in idx_copies(k, slot):
            pltpu.async_copy(a, b, sem_s[slot])

    def wait_idx(k, slot):
        for a, b in idx_copies(k, slot):
            pltpu.make_async_copy(a, b, sem_s[slot]).wait()

    def big_copies(k, slot):
        base = edge0 + k * CHUNK
        return (
            (ep2.at[pl.ds(ep_base0 + base, CHUNK)], ep_buf[slot]),
            (h2.at[src_v[slot]], g_buf[slot]),
        )

    def issue_big(k, slot):
        for a, b in big_copies(k, slot):
            pltpu.async_copy(a, b, sem_b[slot])

    def wait_big(k, slot):
        for a, b in big_copies(k, slot):
            pltpu.make_async_copy(a, b, sem_b[slot]).wait()

    def compute(slot):
        gb = g_buf[slot]
        eb = ep_buf[slot]
        pb = pol_v[slot]

        def edge_body(j2, _):
            for t in range(2):
                j = j2 * 2 + t
                pv = pb[j, :]
                for r in range(DH // 16):
                    sl = pl.ds(r * 16, 16)
                    gb[j, sl] = jnp.maximum(gb[j, sl] + eb[j, sl], 0.0) * pv
            return 0

        lax.fori_loop(0, CHUNK // 2, edge_body, 0)

    def chunk_step(k, s0, s1, s2):
        k2 = jnp.minimum(k + 2, LAST)
        wait_idx(k2, s2)
        issue_big(k2, s2)
        wait_big(k, s0)
        compute(s0)
        pltpu.sync_copy(g_buf[s0], agg_sh.at[dst_v[s0]], add=True)
        issue_idx(jnp.minimum(k + 3, LAST), s0)

    issue_idx(0, 0)
    issue_idx(1, 1)
    wait_idx(0, 0)
    issue_big(0, 0)
    wait_idx(1, 1)
    issue_big(1, 1)
    issue_idx(2, 2)

    def tri_body(j, _):
        chunk_step(3 * j, 0, 1, 2)
        chunk_step(3 * j + 1, 1, 2, 0)
        chunk_step(3 * j + 2, 2, 0, 1)
        return 0

    lax.fori_loop(0, CHUNKS_PER_TILE // 3, tri_body, 0)
    chunk_step(CHUNKS_PER_TILE - 1, 0, 1, 2)

    wait_big(LAST, 1)
    wait_big(LAST, 2)
    wait_idx(LAST, 0)

    plsc.subcore_barrier()
    pltpu.sync_copy(agg_sh.at[pl.ds(row0, ROWS_PER_TILE)],
                    agg_out.at[pl.ds(c * N_PAD + row0, ROWS_PER_TILE)])


@functools.cache
def _make_sc_scatter():
    mesh = plsc.VectorSubcoreMesh(core_axis_name="c", subcore_axis_name="s",
                                  num_cores=2, num_subcores=N_TILES)
    return pl.kernel(
        _sc_scatter_body,
        out_type=jax.ShapeDtypeStruct((2 * N_PAD, DH), jnp.float32),
        mesh=mesh,
        scratch_types=(
            [pltpu.VMEM((CHUNK,), jnp.int32) for _ in range(NSLOT)]
            + [pltpu.VMEM((CHUNK,), jnp.int32) for _ in range(NSLOT)]
            + [pltpu.VMEM((CHUNK, 16), jnp.float32) for _ in range(NSLOT)]
            + [pltpu.VMEM((CHUNK, DH), jnp.float32) for _ in range(NSLOT)]
            + [pltpu.VMEM((CHUNK, DH), jnp.float32) for _ in range(NSLOT)]
            + [pltpu.VMEM_SHARED((AGG_ROWS, DH), jnp.float32)]
            + [pltpu.SemaphoreType.DMA for _ in range(2 * NSLOT)]
        ),
    )


def kernel(x, edge_index, edge_attr, W_msg, W_edge, W_self, b):
    src = edge_index[0]
    dst = edge_index[1]

    x_pad = jnp.concatenate(
        [x, jnp.zeros((N_PAD - N, D), jnp.float32)], axis=0)
    src2 = jnp.concatenate([src, src + N_PAD])
    dst_pad = dst
    b2 = b.reshape(2, DH)
    zeros = jnp.zeros((ROWS_PER_TILE, DH), jnp.float32)

    h2 = pl.pallas_call(
        _h_body,
        grid=(_NB, 2),
        in_specs=[
            pl.BlockSpec((_NROWS, D), lambda i, c: (i, 0)),
            pl.BlockSpec((D, DH), lambda i, c: (0, c)),
        ],
        out_specs=pl.BlockSpec((_NROWS, DH), lambda i, c: (c * _NB + i, 0)),
        out_shape=jax.ShapeDtypeStruct((2 * N_PAD, DH), jnp.float32),
    )(x_pad, W_msg)

    ep2, pol2d = pl.pallas_call(
        _ep_body,
        grid=(_EB, 2),
        in_specs=[
            pl.BlockSpec((_EROWS, DE), lambda i, c: (i, 0)),
            pl.BlockSpec((DE, DH), lambda i, c: (0, c)),
            pl.BlockSpec((2, DH), lambda i, c: (0, 0)),
        ],
        out_specs=[
            pl.BlockSpec((_EROWS, DH), lambda i, c: (c * _EB + i, 0)),
            pl.BlockSpec((_EROWS, 16), lambda i, c: (i, 0)),
        ],
        out_shape=[
            jax.ShapeDtypeStruct((2 * E_PAD, DH), jnp.float32),
            jax.ShapeDtypeStruct((E_PAD, 16), jnp.float32),
        ],
    )(edge_attr, W_edge, b2)

    agg2 = _make_sc_scatter()(h2, src2, dst_pad, pol2d, ep2, zeros)

    out = pl.pallas_call(
        _out_body,
        grid=(_NB,),
        in_specs=[
            pl.BlockSpec((_NROWS, D), lambda i: (i, 0)),
            pl.BlockSpec((D, D), lambda i: (0, 0)),
            pl.BlockSpec((_NROWS, DH), lambda i: (i, 0)),
            pl.BlockSpec((_NROWS, DH), lambda i: (_NB + i, 0)),
        ],
        out_specs=pl.BlockSpec((_NROWS, D), lambda i: (i, 0)),
        out_shape=jax.ShapeDtypeStruct((N_PAD, D), jnp.float32),
    )(x_pad, W_self, agg2, agg2)

    return out[:N]
